# Initial kernel scaffold; baseline (speedup 1.0000x reference)
import jax
import jax.numpy as jnp
from jax.experimental import pallas as pl


def kernel(input, table):
    raise NotImplementedError("write your pallas kernel here")


# speedup vs baseline: 1.0933x; 1.0933x over previous
import functools

import jax
import jax.numpy as jnp
from jax import lax
from jax.experimental import pallas as pl
from jax.experimental.pallas import tpu as pltpu
from jax.experimental.pallas import tpu_sc as plsc

_GROUP = 128
_G_PER_STEP = 8
_CHUNK = _GROUP * _G_PER_STEP


def _embed_lookup(idx2d, table, *, n_rows, embed, num_cores, num_subcores):
    nw = num_cores * num_subcores
    per_w = n_rows // nw
    steps = per_w // _CHUNK
    groups_per_w = per_w // _GROUP

    mesh = plsc.VectorSubcoreMesh(core_axis_name="c", subcore_axis_name="s")

    @functools.partial(
        pl.kernel,
        mesh=mesh,
        compiler_params=pltpu.CompilerParams(use_tc_tiling_on_sc=False),
        out_type=jax.ShapeDtypeStruct((n_rows, embed), jnp.float32),
        scratch_types=[
            pltpu.VMEM((_G_PER_STEP, _GROUP), jnp.int32),
            pltpu.VMEM((_CHUNK, embed), jnp.float32),
            pltpu.SemaphoreType.DMA,
        ],
    )
    def k(idx_hbm, tab_hbm, out_hbm, idx_v, rows_v, sem):
        wid = lax.axis_index("s") * num_cores + lax.axis_index("c")
        g_base = wid * groups_per_w
        r_base = wid * per_w

        def body(step, _):
            pltpu.sync_copy(
                idx_hbm.at[pl.ds(g_base + step * _G_PER_STEP, _G_PER_STEP)],
                idx_v,
            )
            handles = []
            for j in range(_G_PER_STEP):
                handles.append(
                    pltpu.async_copy(
                        tab_hbm.at[idx_v.at[j]],
                        rows_v.at[pl.ds(j * _GROUP, _GROUP)],
                        sem,
                    )
                )
            for h in handles:
                h.wait()
            pltpu.sync_copy(
                rows_v, out_hbm.at[pl.ds(r_base + step * _CHUNK, _CHUNK)]
            )
            return 0

        lax.fori_loop(0, steps, body, 0)

    return k(idx2d, table)


def kernel(input, table):
    batch, hist = input.shape
    vocab, embed = table.shape
    n_rows = batch * hist
    idx2d = input.reshape(n_rows // _GROUP, _GROUP).astype(jnp.int32)

    info = plsc.get_sparse_core_info()
    out = _embed_lookup(
        idx2d,
        table,
        n_rows=n_rows,
        embed=embed,
        num_cores=info.num_cores,
        num_subcores=info.num_subcores,
    )
    return out.reshape(batch, hist, embed)


# speedup vs baseline: 1.0944x; 1.0010x over previous
import functools

import jax
import jax.numpy as jnp
from jax import lax
from jax.experimental import pallas as pl
from jax.experimental.pallas import tpu as pltpu
from jax.experimental.pallas import tpu_sc as plsc

_GROUP = 128
_G_PER_STEP = 8
_CHUNK = _GROUP * _G_PER_STEP


def _embed_lookup(idx2d, table, *, n_rows, embed, num_cores, num_subcores):
    nw = num_cores * num_subcores
    per_w = n_rows // nw
    steps = per_w // _CHUNK
    groups_per_w = per_w // _GROUP

    mesh = plsc.VectorSubcoreMesh(core_axis_name="c", subcore_axis_name="s")

    @functools.partial(
        pl.kernel,
        mesh=mesh,
        compiler_params=pltpu.CompilerParams(use_tc_tiling_on_sc=False),
        out_type=jax.ShapeDtypeStruct((n_rows, embed), jnp.float32),
        scratch_types=[
            pltpu.VMEM((_CHUNK,), jnp.int32),
            pltpu.VMEM((_CHUNK, embed), jnp.float32),
            pltpu.SemaphoreType.DMA,
        ],
    )
    def k(idx_hbm, tab_hbm, out_hbm, idx_v, rows_v, sem):
        wid = lax.axis_index("s") * num_cores + lax.axis_index("c")
        r_base = wid * per_w

        def body(step, _):
            roff = r_base + step * _CHUNK
            pltpu.sync_copy(idx_hbm.at[pl.ds(roff, _CHUNK)], idx_v)
            pltpu.async_copy(tab_hbm.at[idx_v], rows_v, sem).wait()
            pltpu.sync_copy(rows_v, out_hbm.at[pl.ds(roff, _CHUNK)])
            return 0

        lax.fori_loop(0, steps, body, 0)

    return k(idx2d, table)


def kernel(input, table):
    batch, hist = input.shape
    vocab, embed = table.shape
    n_rows = batch * hist
    idx2d = input.reshape(n_rows).astype(jnp.int32)

    info = plsc.get_sparse_core_info()
    out = _embed_lookup(
        idx2d,
        table,
        n_rows=n_rows,
        embed=embed,
        num_cores=info.num_cores,
        num_subcores=info.num_subcores,
    )
    return out.reshape(batch, hist, embed)
